# baseline (speedup 1.0000x reference)
import jax
import jax.numpy as jnp
from jax import lax
from jax.experimental import pallas as pl
from jax.experimental.pallas import tpu as pltpu

_N = 2048
_V = 32000
_C = 8
_F = 4
_SLOTS = _C * _F
_LANES = 128
_K = _SLOTS * _LANES


def _body(tiles_ref, nu_ref, sel_ref, logits_ref, fl_ref, mlm_ref, lab_ref,
          loss_ref, pred_ref, comp_ref, sem_ref):
    nu = nu_ref[0]

    def _copy(u):
        t = tiles_ref[u]
        return pltpu.make_async_copy(
            logits_ref.at[:, pl.ds(t * _LANES, _LANES)],
            comp_ref.at[:, pl.ds(u * _LANES, _LANES)],
            sem_ref,
        )

    for u in range(_SLOTS):
        @pl.when(u < nu)
        def _start(u=u):
            _copy(u).start()

        @pl.when(u >= nu)
        def _pad(u=u):
            comp_ref[:, pl.ds(u * _LANES, _LANES)] = jnp.zeros(
                (_N, _LANES), jnp.float32)

    for u in range(_SLOTS):
        @pl.when(u < nu)
        def _drain(u=u):
            _copy(u).wait()

    scores_t = lax.dot_general(
        sel_ref[...], comp_ref[...],
        (((1,), (1,)), ((), ())),
        precision=lax.Precision.HIGHEST,
        preferred_element_type=jnp.float32,
    )

    mask = mlm_ref[...] >= 0
    fl = fl_ref[...]
    scores = jnp.where(mask, scores_t / fl, 0.0)

    best = scores[0:1, :]
    pred = jnp.zeros((1, _N), jnp.int32)
    for cc in range(1, _C):
        row = scores[cc:cc + 1, :]
        upd = row > best
        best = jnp.where(upd, row, best)
        pred = jnp.where(upd, cc, pred)

    se = jnp.zeros((1, _N), jnp.float32)
    for cc in range(_C):
        se = se + jnp.exp(scores[cc:cc + 1, :] - best)
    lse = jnp.log(se) + best

    lab = lab_ref[...]
    s_lab = jnp.zeros((1, _N), jnp.float32)
    for cc in range(_C):
        s_lab = s_lab + jnp.where(lab == cc, scores[cc:cc + 1, :], 0.0)

    loss_ref[0, 0] = jnp.sum(lse - s_lab) / float(_N)
    pred_ref[...] = pred


def kernel(logits, mlm_labels, labels, weight, m2c, filler_len):
    logits2d = logits.reshape(_N, _V)
    fidx = jnp.maximum(m2c.reshape(-1), 0).astype(jnp.int32)
    tile = fidx // _LANES
    lane = fidx % _LANES
    order = jnp.argsort(tile).astype(jnp.int32)
    tiles_sorted = tile[order]
    lanes_sorted = lane[order]
    cs = order // _F
    fs = order % _F
    is_new = jnp.concatenate([
        jnp.ones((1,), jnp.int32),
        (tiles_sorted[1:] != tiles_sorted[:-1]).astype(jnp.int32),
    ])
    upos = jnp.cumsum(is_new).astype(jnp.int32) - 1
    n_unique = upos[-1] + 1
    tiles_u = jnp.zeros((_SLOTS,), jnp.int32).at[upos].set(tiles_sorted)
    wk = (weight.reshape(-1)[order]
          * (m2c.reshape(-1)[order] > 0).astype(jnp.float32))
    selflat = jnp.zeros((_C, _K), jnp.float32).at[
        cs, upos * _LANES + lanes_sorted].add(wk)

    loss, pred = pl.pallas_call(
        _body,
        in_specs=[
            pl.BlockSpec(memory_space=pltpu.SMEM),
            pl.BlockSpec(memory_space=pltpu.SMEM),
            pl.BlockSpec(memory_space=pltpu.VMEM),
            pl.BlockSpec(memory_space=pltpu.HBM),
            pl.BlockSpec(memory_space=pltpu.VMEM),
            pl.BlockSpec(memory_space=pltpu.VMEM),
            pl.BlockSpec(memory_space=pltpu.VMEM),
        ],
        out_specs=[
            pl.BlockSpec(memory_space=pltpu.SMEM),
            pl.BlockSpec(memory_space=pltpu.VMEM),
        ],
        out_shape=[
            jax.ShapeDtypeStruct((1, 1), jnp.float32),
            jax.ShapeDtypeStruct((1, _N), jnp.int32),
        ],
        scratch_shapes=[
            pltpu.VMEM((_N, _K), jnp.float32),
            pltpu.SemaphoreType.DMA,
        ],
    )(
        tiles_u,
        n_unique.reshape(1),
        selflat,
        logits2d,
        filler_len.reshape(_C, 1),
        mlm_labels.reshape(1, _N),
        labels.reshape(1, _N).astype(jnp.int32),
    )
    return loss[0, 0], pred.reshape(_N)


# speedup vs baseline: 1.7540x; 1.7540x over previous
import jax
import jax.numpy as jnp
from jax import lax
from jax.experimental import pallas as pl
from jax.experimental.pallas import tpu as pltpu

_N = 2048
_V = 32000
_C = 8
_F = 4
_SLOTS = _C * _F
_LANES = 128


def _body(tiles_ref, upos_ref, cs_ref, fs_ref, lanes_ref,
          logits_ref, w_ref, m2c_ref, fl_ref, mlm_ref, lab_ref,
          loss_ref, pred_ref, acc_ref):
    s = pl.program_id(0)

    @pl.when(s == 0)
    def _init():
        acc_ref[...] = jnp.zeros((_C, _N), jnp.float32)

    row_i = lax.broadcasted_iota(jnp.int32, (_C, _LANES), 0)
    lane_i = lax.broadcasted_iota(jnp.int32, (_C, _LANES), 1)
    sel = jnp.zeros((_C, _LANES), jnp.float32)
    for j in range(_SLOTS):
        c = cs_ref[j]
        f = fs_ref[j]
        keep = (m2c_ref[c, f] > 0).astype(jnp.float32)
        wk = w_ref[c, f] * keep
        own = (upos_ref[j] == s).astype(jnp.float32)
        hit = ((row_i == c) & (lane_i == lanes_ref[j])).astype(jnp.float32)
        sel = sel + hit * (wk * own)
    acc_ref[...] += lax.dot_general(
        sel, logits_ref[...],
        (((1,), (1,)), ((), ())),
        precision=lax.Precision.HIGHEST,
        preferred_element_type=jnp.float32,
    )

    @pl.when(s == pl.num_programs(0) - 1)
    def _finish():
        mask = mlm_ref[...] >= 0
        fl = fl_ref[...]
        scores = jnp.where(mask, acc_ref[...] / fl, 0.0)

        best = scores[0:1, :]
        pred = jnp.zeros((1, _N), jnp.int32)
        for cc in range(1, _C):
            row = scores[cc:cc + 1, :]
            upd = row > best
            best = jnp.where(upd, row, best)
            pred = jnp.where(upd, cc, pred)

        se = jnp.zeros((1, _N), jnp.float32)
        for cc in range(_C):
            se = se + jnp.exp(scores[cc:cc + 1, :] - best)
        lse = jnp.log(se) + best

        lab = lab_ref[...]
        s_lab = jnp.zeros((1, _N), jnp.float32)
        for cc in range(_C):
            s_lab = s_lab + jnp.where(lab == cc, scores[cc:cc + 1, :], 0.0)

        loss_ref[0, 0] = jnp.sum(lse - s_lab) / float(_N)
        pred_ref[...] = pred


def kernel(logits, mlm_labels, labels, weight, m2c, filler_len):
    logits2d = logits.reshape(_N, _V)
    fidx = jnp.maximum(m2c.reshape(-1), 0).astype(jnp.int32)
    tile = fidx // _LANES
    lane = fidx % _LANES
    order = jnp.argsort(tile).astype(jnp.int32)
    tiles_sorted = tile[order]
    lanes_sorted = lane[order]
    cs = order // _F
    fs = order % _F
    is_new = jnp.concatenate([
        jnp.ones((1,), jnp.int32),
        (tiles_sorted[1:] != tiles_sorted[:-1]).astype(jnp.int32),
    ])
    upos = jnp.cumsum(is_new).astype(jnp.int32) - 1
    n_unique = upos[-1] + 1
    tiles_u = jnp.zeros((_SLOTS,), jnp.int32).at[upos].set(tiles_sorted)

    grid_spec = pltpu.PrefetchScalarGridSpec(
        num_scalar_prefetch=5,
        grid=(n_unique,),
        in_specs=[
            pl.BlockSpec((_N, _LANES), lambda s, T, U, C, F, L: (0, T[s])),
            pl.BlockSpec(memory_space=pltpu.SMEM),
            pl.BlockSpec(memory_space=pltpu.SMEM),
            pl.BlockSpec(memory_space=pltpu.VMEM),
            pl.BlockSpec(memory_space=pltpu.VMEM),
            pl.BlockSpec(memory_space=pltpu.VMEM),
        ],
        out_specs=[
            pl.BlockSpec(memory_space=pltpu.SMEM),
            pl.BlockSpec(memory_space=pltpu.VMEM),
        ],
        scratch_shapes=[pltpu.VMEM((_C, _N), jnp.float32)],
    )

    loss, pred = pl.pallas_call(
        _body,
        grid_spec=grid_spec,
        out_shape=[
            jax.ShapeDtypeStruct((1, 1), jnp.float32),
            jax.ShapeDtypeStruct((1, _N), jnp.int32),
        ],
    )(
        tiles_u, upos, cs, fs, lanes_sorted,
        logits2d,
        weight,
        m2c,
        filler_len.reshape(_C, 1),
        mlm_labels.reshape(1, _N),
        labels.reshape(1, _N).astype(jnp.int32),
    )
    return loss[0, 0], pred.reshape(_N)


# speedup vs baseline: 2.0111x; 1.1466x over previous
import jax
import jax.numpy as jnp
from jax import lax
from jax.experimental import pallas as pl
from jax.experimental.pallas import tpu as pltpu

_N = 2048
_V = 32000
_C = 8
_F = 4
_SLOTS = _C * _F
_LANES = 128


def _body(tiles_ref, upos_ref, lanes_ref, wk_ref,
          logits_ref, fl_ref, mlm_ref, lab_ref,
          loss_ref, pred_ref, acc_ref):
    s = pl.program_id(0)

    @pl.when(s == 0)
    def _init():
        acc_ref[...] = jnp.zeros((_C, _N), jnp.float32)

    row_i = lax.broadcasted_iota(jnp.int32, (_C, _LANES), 0)
    lane_i = lax.broadcasted_iota(jnp.int32, (_C, _LANES), 1)
    sel = jnp.zeros((_C, _LANES), jnp.float32)
    for j in range(_SLOTS):
        own = (upos_ref[0, j] == s).astype(jnp.float32)
        hit = ((row_i == j // _F)
               & (lane_i == lanes_ref[0, j])).astype(jnp.float32)
        sel = sel + hit * (wk_ref[0, j] * own)
    acc_ref[...] += lax.dot_general(
        sel, logits_ref[...],
        (((1,), (1,)), ((), ())),
        precision=lax.Precision.HIGHEST,
        preferred_element_type=jnp.float32,
    )

    @pl.when(s == pl.num_programs(0) - 1)
    def _finish():
        mask = mlm_ref[...] >= 0
        fl = fl_ref[...]
        scores = jnp.where(mask, acc_ref[...] / fl, 0.0)

        best = scores[0:1, :]
        pred = jnp.zeros((1, _N), jnp.int32)
        for cc in range(1, _C):
            row = scores[cc:cc + 1, :]
            upd = row > best
            best = jnp.where(upd, row, best)
            pred = jnp.where(upd, cc, pred)

        se = jnp.zeros((1, _N), jnp.float32)
        for cc in range(_C):
            se = se + jnp.exp(scores[cc:cc + 1, :] - best)
        lse = jnp.log(se) + best

        lab = lab_ref[...]
        s_lab = jnp.zeros((1, _N), jnp.float32)
        for cc in range(_C):
            s_lab = s_lab + jnp.where(lab == cc, scores[cc:cc + 1, :], 0.0)

        loss_ref[0, 0] = jnp.sum(lse - s_lab) / float(_N)
        pred_ref[...] = pred


def kernel(logits, mlm_labels, labels, weight, m2c, filler_len):
    logits2d = logits.reshape(_N, _V)
    fidx = jnp.maximum(m2c.reshape(-1), 0).astype(jnp.int32)
    tile = fidx // _LANES
    lane = fidx % _LANES
    eq = tile[None, :] == tile[:, None]
    firstocc = jnp.argmax(eq, axis=1).astype(jnp.int32)
    is_first = (firstocc == jnp.arange(_SLOTS, dtype=jnp.int32)).astype(
        jnp.int32)
    ranks = jnp.cumsum(is_first).astype(jnp.int32) - 1
    upos = ranks[firstocc]
    n_unique = ranks[-1] + 1
    tiles_u = jnp.zeros((_SLOTS,), jnp.int32).at[upos].set(tile)
    wk = (weight.reshape(-1)
          * (m2c.reshape(-1) > 0).astype(jnp.float32))

    grid_spec = pltpu.PrefetchScalarGridSpec(
        num_scalar_prefetch=1,
        grid=(n_unique,),
        in_specs=[
            pl.BlockSpec(memory_space=pltpu.SMEM),
            pl.BlockSpec(memory_space=pltpu.SMEM),
            pl.BlockSpec(memory_space=pltpu.SMEM),
            pl.BlockSpec((_N, _LANES), lambda s, T: (0, T[s])),
            pl.BlockSpec(memory_space=pltpu.VMEM),
            pl.BlockSpec(memory_space=pltpu.VMEM),
            pl.BlockSpec(memory_space=pltpu.VMEM),
        ],
        out_specs=[
            pl.BlockSpec(memory_space=pltpu.SMEM),
            pl.BlockSpec(memory_space=pltpu.VMEM),
        ],
        scratch_shapes=[pltpu.VMEM((_C, _N), jnp.float32)],
    )

    loss, pred = pl.pallas_call(
        _body,
        grid_spec=grid_spec,
        out_shape=[
            jax.ShapeDtypeStruct((1, 1), jnp.float32),
            jax.ShapeDtypeStruct((1, _N), jnp.int32),
        ],
    )(
        tiles_u,
        upos.reshape(1, _SLOTS),
        lane.reshape(1, _SLOTS),
        wk.reshape(1, _SLOTS),
        logits2d,
        filler_len.reshape(_C, 1),
        mlm_labels.reshape(1, _N),
        labels.reshape(1, _N).astype(jnp.int32),
    )
    return loss[0, 0], pred.reshape(_N)


# speedup vs baseline: 2.1616x; 1.0748x over previous
import jax
import jax.numpy as jnp
from jax import lax
from jax.experimental import pallas as pl
from jax.experimental.pallas import tpu as pltpu

_N = 2048
_V = 32000
_C = 8
_F = 4
_SLOTS = _C * _F
_LANES = 128


def _body(tiles_ref, upos_ref, lanes_ref, wk_ref,
          logits_a_ref, logits_b_ref, fl_ref, mlm_ref, lab_ref,
          loss_ref, pred_ref, acc_ref):
    s = pl.program_id(0)

    @pl.when(s == 0)
    def _init():
        acc_ref[...] = jnp.zeros((_C, _N), jnp.float32)

    row_i = lax.broadcasted_iota(jnp.int32, (_C, _LANES), 0)
    lane_i = lax.broadcasted_iota(jnp.int32, (_C, _LANES), 1)
    for half, blk in ((0, logits_a_ref), (1, logits_b_ref)):
        sel = jnp.zeros((_C, _LANES), jnp.float32)
        for j in range(_SLOTS):
            own = (upos_ref[0, j] == 2 * s + half).astype(jnp.float32)
            hit = ((row_i == j // _F)
                   & (lane_i == lanes_ref[0, j])).astype(jnp.float32)
            sel = sel + hit * (wk_ref[0, j] * own)
        acc_ref[...] += lax.dot_general(
            sel, blk[...],
            (((1,), (1,)), ((), ())),
            precision=lax.Precision.HIGHEST,
            preferred_element_type=jnp.float32,
        )

    @pl.when(s == pl.num_programs(0) - 1)
    def _finish():
        mask = mlm_ref[...] >= 0
        fl = fl_ref[...]
        scores = jnp.where(mask, acc_ref[...] / fl, 0.0)

        best = scores[0:1, :]
        pred = jnp.zeros((1, _N), jnp.int32)
        for cc in range(1, _C):
            row = scores[cc:cc + 1, :]
            upd = row > best
            best = jnp.where(upd, row, best)
            pred = jnp.where(upd, cc, pred)

        se = jnp.zeros((1, _N), jnp.float32)
        for cc in range(_C):
            se = se + jnp.exp(scores[cc:cc + 1, :] - best)
        lse = jnp.log(se) + best

        lab = lab_ref[...]
        s_lab = jnp.zeros((1, _N), jnp.float32)
        for cc in range(_C):
            s_lab = s_lab + jnp.where(lab == cc, scores[cc:cc + 1, :], 0.0)

        loss_ref[0, 0] = jnp.sum(lse - s_lab) / float(_N)
        pred_ref[...] = pred


def kernel(logits, mlm_labels, labels, weight, m2c, filler_len):
    logits2d = logits.reshape(_N, _V)
    fidx = jnp.maximum(m2c.reshape(-1), 0).astype(jnp.int32)
    tile = fidx // _LANES
    lane = fidx % _LANES
    eq = tile[None, :] == tile[:, None]
    firstocc = jnp.argmax(eq, axis=1).astype(jnp.int32)
    is_first = (firstocc == jnp.arange(_SLOTS, dtype=jnp.int32)).astype(
        jnp.int32)
    ranks = jnp.cumsum(is_first).astype(jnp.int32) - 1
    upos = ranks[firstocc]
    n_unique = ranks[-1] + 1
    tiles_u = jnp.zeros((_SLOTS,), jnp.int32).at[upos].set(tile)
    wk = (weight.reshape(-1)
          * (m2c.reshape(-1) > 0).astype(jnp.float32))

    grid_spec = pltpu.PrefetchScalarGridSpec(
        num_scalar_prefetch=1,
        grid=((n_unique + 1) // 2,),
        in_specs=[
            pl.BlockSpec(memory_space=pltpu.SMEM),
            pl.BlockSpec(memory_space=pltpu.SMEM),
            pl.BlockSpec(memory_space=pltpu.SMEM),
            pl.BlockSpec((_N, _LANES), lambda s, T: (0, T[2 * s])),
            pl.BlockSpec((_N, _LANES), lambda s, T: (0, T[2 * s + 1])),
            pl.BlockSpec(memory_space=pltpu.VMEM),
            pl.BlockSpec(memory_space=pltpu.VMEM),
            pl.BlockSpec(memory_space=pltpu.VMEM),
        ],
        out_specs=[
            pl.BlockSpec(memory_space=pltpu.SMEM),
            pl.BlockSpec(memory_space=pltpu.VMEM),
        ],
        scratch_shapes=[pltpu.VMEM((_C, _N), jnp.float32)],
    )

    loss, pred = pl.pallas_call(
        _body,
        grid_spec=grid_spec,
        out_shape=[
            jax.ShapeDtypeStruct((1, 1), jnp.float32),
            jax.ShapeDtypeStruct((1, _N), jnp.int32),
        ],
    )(
        tiles_u,
        upos.reshape(1, _SLOTS),
        lane.reshape(1, _SLOTS),
        wk.reshape(1, _SLOTS),
        logits2d,
        logits2d,
        filler_len.reshape(_C, 1),
        mlm_labels.reshape(1, _N),
        labels.reshape(1, _N).astype(jnp.int32),
    )
    return loss[0, 0], pred.reshape(_N)
